# baseline (speedup 1.0000x reference)
import functools

import jax
import jax.numpy as jnp
from jax.experimental import pallas as pl
from jax.experimental.pallas import tpu as pltpu
from jax.experimental.pallas import tpu_sc as plsc

B = 8
N = 8192
G = 256
K = 32
_INF = float("inf")

_SC_NC = 2
_SC_NS = 16
_SC_NW = _SC_NC * _SC_NS
_ROWS = B * G * K
_RPT = _ROWS // _SC_NW
_CPT = _RPT // K
_IDX_SUB = _RPT // 128


def _fps_kernel(x_ref, y_ref, z_ref, cx_ref, cy_ref, cz_ref):
    f32 = jnp.float32
    x = x_ref[:, :]
    y = y_ref[:, :]
    z = z_ref[:, :]
    iota_bn = jax.lax.broadcasted_iota(jnp.int32, (B, N), 1)
    iota_bg = jax.lax.broadcasted_iota(jnp.int32, (B, G), 1)

    def fps_body(i, carry):
        dmin, far, cxa, cya, cza = carry
        sel = iota_bn == far
        cxi = jnp.sum(jnp.where(sel, x, 0.0), axis=1, keepdims=True)
        cyi = jnp.sum(jnp.where(sel, y, 0.0), axis=1, keepdims=True)
        czi = jnp.sum(jnp.where(sel, z, 0.0), axis=1, keepdims=True)
        colm = iota_bg == i
        cxa = jnp.where(colm, cxi, cxa)
        cya = jnp.where(colm, cyi, cya)
        cza = jnp.where(colm, czi, cza)
        dx = x - cxi
        dy = y - cyi
        dz = z - czi
        d = (dx * dx + dy * dy) + dz * dz
        dmin = jnp.minimum(dmin, d)
        mx = jnp.max(dmin, axis=1, keepdims=True)
        far = jnp.min(jnp.where(dmin == mx, iota_bn, N), axis=1, keepdims=True)
        return dmin, far, cxa, cya, cza

    dmin0 = jnp.full((B, N), _INF, dtype=f32)
    far0 = jnp.zeros((B, 1), dtype=jnp.int32)
    ca0 = jnp.zeros((B, G), dtype=f32)
    _, _, cxa, cya, cza = jax.lax.fori_loop(
        0, G, fps_body, (dmin0, far0, ca0, ca0, ca0))
    cx_ref[:, :] = cxa
    cy_ref[:, :] = cya
    cz_ref[:, :] = cza


def _group_kernel(x_ref, y_ref, z_ref, cxr, cyr, czr, dmat_ref,
                  idx_ref, cx_ref, cy_ref, cz_ref):
    f32 = jnp.float32
    cxa = cxr[:, :]
    cya = cyr[:, :]
    cza = czr[:, :]
    iota_bg = jax.lax.broadcasted_iota(jnp.int32, (B, G), 1)

    iota_bgg1 = jax.lax.broadcasted_iota(jnp.int32, (B, G, G), 1)
    iota_bgg2 = jax.lax.broadcasted_iota(jnp.int32, (B, G, G), 2)
    dmat = jnp.where(iota_bgg1 == iota_bgg2, _INF, dmat_ref[:, :, :])

    def chain_body(s, carry):
        visited, cur, orda = carry
        rowm = iota_bgg1 == cur[:, :, None]
        row = jnp.sum(jnp.where(rowm, dmat, 0.0), axis=1)
        row = jnp.where(visited != 0, _INF, row)
        mn = jnp.min(row, axis=1, keepdims=True)
        nxt = jnp.min(jnp.where(row == mn, iota_bg, G), axis=1, keepdims=True)
        orda = jnp.where(iota_bg == s, nxt, orda)
        visited = jnp.maximum(visited, (iota_bg == nxt).astype(jnp.int32))
        return visited, nxt, orda

    visited0 = (iota_bg == 0).astype(jnp.int32)
    cur0 = jnp.zeros((B, 1), dtype=jnp.int32)
    order0 = jnp.zeros((B, G), dtype=jnp.int32)
    _, _, order = jax.lax.fori_loop(
        1, G, chain_body, (visited0, cur0, order0))

    iota_gn = jax.lax.broadcasted_iota(jnp.int32, (G, N), 1)
    iota_gk = jax.lax.broadcasted_iota(jnp.int32, (G, K), 1)
    iota_gg2 = jax.lax.broadcasted_iota(jnp.int32, (G, G), 1)
    iota_b1 = jax.lax.broadcasted_iota(jnp.int32, (B, 1), 0)
    hi_prec = jax.lax.Precision.HIGHEST

    def knn_batch(b, _):
        sel = iota_b1 == b
        cxb = jnp.transpose(jnp.sum(jnp.where(sel, cxa, 0.0), axis=0,
                                    keepdims=True))
        cyb = jnp.transpose(jnp.sum(jnp.where(sel, cya, 0.0), axis=0,
                                    keepdims=True))
        czb = jnp.transpose(jnp.sum(jnp.where(sel, cza, 0.0), axis=0,
                                    keepdims=True))
        ordb = jnp.transpose(jnp.sum(jnp.where(sel, order, 0), axis=0,
                                     keepdims=True))
        xb = x_ref[pl.ds(b, 1), :]
        yb = y_ref[pl.ds(b, 1), :]
        zb = z_ref[pl.ds(b, 1), :]
        cb = jnp.concatenate([cxb, cyb, czb], axis=1)
        xm = jnp.concatenate([xb, yb, zb], axis=0)
        ab = jax.lax.dot_general(cb, xm, (((1,), (0,)), ((), ())),
                                 preferred_element_type=f32)
        a2 = jnp.sum(cb * cb, axis=1, keepdims=True)
        b2 = xb * xb + yb * yb + zb * zb
        d2 = a2 + b2 - 2.0 * ab

        def sel_body(k, carry):
            mnp, idp, rem, idacc = carry
            mn_cheap = jnp.min(jnp.where(d2 > mnp, d2, _INF), axis=1,
                               keepdims=True)
            tiec = rem > 0
            mn = jnp.where(tiec, mnp, mn_cheap)
            cand = (d2 == mn) & ((~tiec) | (iota_gn > idp))
            idx = jnp.min(jnp.where(cand, iota_gn, N), axis=1, keepdims=True)
            cnt = jnp.sum(cand.astype(jnp.int32), axis=1, keepdims=True)
            idacc = jnp.where(iota_gk == k, idx, idacc)
            return mn, idx, cnt - 1, idacc

        mn0 = jnp.full((G, 1), -_INF, dtype=f32)
        id0 = jnp.full((G, 1), -1, dtype=jnp.int32)
        rem0 = jnp.zeros((G, 1), dtype=jnp.int32)
        ia0 = jnp.zeros((G, K), dtype=jnp.int32)
        _, _, _, idacc = jax.lax.fori_loop(0, K, sel_body,
                                           (mn0, id0, rem0, ia0))

        pb = (ordb == iota_gg2).astype(f32)
        dot = functools.partial(jax.lax.dot_general,
                                dimension_numbers=(((1,), (0,)), ((), ())),
                                preferred_element_type=f32,
                                precision=hi_prec)
        pidx = dot(pb, idacc.astype(f32)).astype(jnp.int32) + b * N
        idx_ref[pl.ds(b, 1)] = pidx[None]
        cx_ref[pl.ds(b, 1)] = jnp.transpose(dot(pb, cxb))
        cy_ref[pl.ds(b, 1)] = jnp.transpose(dot(pb, cyb))
        cz_ref[pl.ds(b, 1)] = jnp.transpose(dot(pb, czb))
        return 0

    jax.lax.fori_loop(0, B, knn_batch, 0)


def _sc_gather_body(xyzp_hbm, idx_hbm, cent_hbm, out_hbm,
                    idx_v, rows_v, cent_v, sem):
    wid = jax.lax.axis_index("s") * _SC_NC + jax.lax.axis_index("c")
    base = wid * _RPT
    pltpu.sync_copy(idx_hbm.at[pl.ds(wid * _IDX_SUB, _IDX_SUB)], idx_v)
    cps = []
    for j in range(_IDX_SUB):
        cps.append(pltpu.async_copy(xyzp_hbm.at[idx_v.at[j]],
                                    rows_v.at[pl.ds(j * 128, 128)], sem))
    for cp in cps:
        cp.wait()
    pltpu.sync_copy(cent_hbm.at[pl.ds(wid * _CPT, _CPT)], cent_v)

    def cbody(ci, _):
        cv = cent_v[ci]

        def rbody(i, __):
            r = ci * K + i
            rows_v[r] = rows_v[r] - cv
            return 0

        jax.lax.fori_loop(0, K, rbody, 0)
        return 0

    jax.lax.fori_loop(0, _CPT, cbody, 0)
    pltpu.sync_copy(rows_v, out_hbm.at[pl.ds(base, _RPT)])


@jax.jit
def kernel(xyz):
    f32 = jnp.float32
    x = xyz[:, :, 0]
    y = xyz[:, :, 1]
    z = xyz[:, :, 2]

    cxa, cya, cza = pl.pallas_call(
        _fps_kernel,
        out_shape=(
            jax.ShapeDtypeStruct((B, G), f32),
            jax.ShapeDtypeStruct((B, G), f32),
            jax.ShapeDtypeStruct((B, G), f32),
        ),
    )(x, y, z)

    center = jnp.stack([cxa, cya, cza], axis=-1)
    a2 = jnp.sum(center * center, axis=-1)
    ab = jnp.einsum('bmd,bnd->bmn', center, center)
    dmat = a2[:, :, None] + a2[:, None, :] - 2.0 * ab

    idxp, cxp, cyp, czp = pl.pallas_call(
        _group_kernel,
        out_shape=(
            jax.ShapeDtypeStruct((B, G, K), jnp.int32),
            jax.ShapeDtypeStruct((B, G), f32),
            jax.ShapeDtypeStruct((B, G), f32),
            jax.ShapeDtypeStruct((B, G), f32),
        ),
    )(x, y, z, cxa, cya, cza, dmat)

    centp = jnp.stack([cxp, cyp, czp], axis=-1)
    xyzp = jnp.pad(xyz.reshape(B * N, 3), ((0, 0), (0, 13)))
    cent16 = jnp.pad(centp.reshape(B * G, 3), ((0, 0), (0, 13)))
    idx2d = idxp.reshape(_SC_NW * _IDX_SUB, 128)

    nb16 = pl.kernel(
        _sc_gather_body,
        out_type=jax.ShapeDtypeStruct((_ROWS, 16), f32),
        mesh=plsc.VectorSubcoreMesh(core_axis_name="c", subcore_axis_name="s"),
        scratch_types=[
            pltpu.VMEM((_IDX_SUB, 128), jnp.int32),
            pltpu.VMEM((_RPT, 16), f32),
            pltpu.VMEM((_CPT, 16), f32),
            pltpu.SemaphoreType.DMA,
        ],
        compiler_params=pltpu.CompilerParams(use_tc_tiling_on_sc=False),
    )(xyzp, idx2d, cent16)

    neighborhood = nb16.reshape(B, G, K, 16)[:, :, :, 0:3]
    return neighborhood, centp


# speedup vs baseline: 1.0665x; 1.0665x over previous
import functools

import jax
import jax.numpy as jnp
from jax.experimental import pallas as pl
from jax.experimental.pallas import tpu as pltpu
from jax.experimental.pallas import tpu_sc as plsc

B = 8
N = 8192
G = 256
K = 32
_INF = float("inf")

_SC_NC = 2
_SC_NS = 16
_SC_NW = _SC_NC * _SC_NS
_ROWS = B * G * K
_RPT = _ROWS // _SC_NW
_CPT = _RPT // K
_IDX_SUB = _RPT // 128


def _fps_kernel(x_ref, y_ref, z_ref, cx_ref, cy_ref, cz_ref):
    f32 = jnp.float32
    x = x_ref[:, :]
    y = y_ref[:, :]
    z = z_ref[:, :]
    iota_bn = jax.lax.broadcasted_iota(jnp.int32, (B, N), 1)
    iota_bg = jax.lax.broadcasted_iota(jnp.int32, (B, G), 1)

    def fps_body(i, carry):
        dmin, far, cxa, cya, cza = carry
        sel = iota_bn == far
        cxi = jnp.sum(jnp.where(sel, x, 0.0), axis=1, keepdims=True)
        cyi = jnp.sum(jnp.where(sel, y, 0.0), axis=1, keepdims=True)
        czi = jnp.sum(jnp.where(sel, z, 0.0), axis=1, keepdims=True)
        colm = iota_bg == i
        cxa = jnp.where(colm, cxi, cxa)
        cya = jnp.where(colm, cyi, cya)
        cza = jnp.where(colm, czi, cza)
        dx = x - cxi
        dy = y - cyi
        dz = z - czi
        d = (dx * dx + dy * dy) + dz * dz
        dmin = jnp.minimum(dmin, d)
        mx = jnp.max(dmin, axis=1, keepdims=True)
        far = jnp.min(jnp.where(dmin == mx, iota_bn, N), axis=1, keepdims=True)
        return dmin, far, cxa, cya, cza

    dmin0 = jnp.full((B, N), _INF, dtype=f32)
    far0 = jnp.zeros((B, 1), dtype=jnp.int32)
    ca0 = jnp.zeros((B, G), dtype=f32)
    _, _, cxa, cya, cza = jax.lax.fori_loop(
        0, G, fps_body, (dmin0, far0, ca0, ca0, ca0))
    cx_ref[:, :] = cxa
    cy_ref[:, :] = cya
    cz_ref[:, :] = cza


def _group_kernel(x_ref, y_ref, z_ref, cxr, cyr, czr, dmat_ref,
                  idx_ref, cx_ref, cy_ref, cz_ref):
    f32 = jnp.float32
    cxa = cxr[:, :]
    cya = cyr[:, :]
    cza = czr[:, :]
    iota_bg = jax.lax.broadcasted_iota(jnp.int32, (B, G), 1)

    iota_bgg1 = jax.lax.broadcasted_iota(jnp.int32, (B, G, G), 1)
    iota_bgg2 = jax.lax.broadcasted_iota(jnp.int32, (B, G, G), 2)
    dmat = jnp.where(iota_bgg1 == iota_bgg2, _INF, dmat_ref[:, :, :])

    def chain_body(s, carry):
        visited, cur, orda = carry
        rowm = iota_bgg1 == cur[:, :, None]
        row = jnp.sum(jnp.where(rowm, dmat, 0.0), axis=1)
        row = jnp.where(visited != 0, _INF, row)
        mn = jnp.min(row, axis=1, keepdims=True)
        nxt = jnp.min(jnp.where(row == mn, iota_bg, G), axis=1, keepdims=True)
        orda = jnp.where(iota_bg == s, nxt, orda)
        visited = jnp.maximum(visited, (iota_bg == nxt).astype(jnp.int32))
        return visited, nxt, orda

    visited0 = (iota_bg == 0).astype(jnp.int32)
    cur0 = jnp.zeros((B, 1), dtype=jnp.int32)
    order0 = jnp.zeros((B, G), dtype=jnp.int32)
    _, _, order = jax.lax.fori_loop(
        1, G, chain_body, (visited0, cur0, order0))

    iota_gn = jax.lax.broadcasted_iota(jnp.int32, (G, N), 1)
    iota_gk = jax.lax.broadcasted_iota(jnp.int32, (G, K), 1)
    iota_gg2 = jax.lax.broadcasted_iota(jnp.int32, (G, G), 1)
    iota_b1 = jax.lax.broadcasted_iota(jnp.int32, (B, 1), 0)
    hi_prec = jax.lax.Precision.HIGHEST

    def _prep(b):
        sel = iota_b1 == b
        cxb = jnp.transpose(jnp.sum(jnp.where(sel, cxa, 0.0), axis=0,
                                    keepdims=True))
        cyb = jnp.transpose(jnp.sum(jnp.where(sel, cya, 0.0), axis=0,
                                    keepdims=True))
        czb = jnp.transpose(jnp.sum(jnp.where(sel, cza, 0.0), axis=0,
                                    keepdims=True))
        ordb = jnp.transpose(jnp.sum(jnp.where(sel, order, 0), axis=0,
                                     keepdims=True))
        xb = x_ref[pl.ds(b, 1), :]
        yb = y_ref[pl.ds(b, 1), :]
        zb = z_ref[pl.ds(b, 1), :]
        cb = jnp.concatenate([cxb, cyb, czb], axis=1)
        xm = jnp.concatenate([xb, yb, zb], axis=0)
        ab = jax.lax.dot_general(cb, xm, (((1,), (0,)), ((), ())),
                                 preferred_element_type=f32)
        a2 = jnp.sum(cb * cb, axis=1, keepdims=True)
        b2 = xb * xb + yb * yb + zb * zb
        d2 = a2 + b2 - 2.0 * ab
        return d2, cxb, cyb, czb, ordb

    def _sel_step(d2, mnp, idp):
        after = (d2 > mnp) | ((d2 == mnp) & (iota_gn > idp))
        mn = jnp.min(jnp.where(after, d2, _INF), axis=1, keepdims=True)
        tie = mn == mnp
        cand = (d2 == mn) & ((~tie) | (iota_gn > idp))
        idx = jnp.min(jnp.where(cand, iota_gn, N), axis=1, keepdims=True)
        return mn, idx

    def _finish(b, idacc, cxb, cyb, czb, ordb):
        pb = (ordb == iota_gg2).astype(f32)
        dot = functools.partial(jax.lax.dot_general,
                                dimension_numbers=(((1,), (0,)), ((), ())),
                                preferred_element_type=f32,
                                precision=hi_prec)
        pidx = dot(pb, idacc.astype(f32)).astype(jnp.int32) + b * N
        idx_ref[pl.ds(b, 1)] = pidx[None]
        cx_ref[pl.ds(b, 1)] = jnp.transpose(dot(pb, cxb))
        cy_ref[pl.ds(b, 1)] = jnp.transpose(dot(pb, cyb))
        cz_ref[pl.ds(b, 1)] = jnp.transpose(dot(pb, czb))

    def knn_pair(p, _):
        b0 = 2 * p
        b1 = 2 * p + 1
        d2a, cxb0, cyb0, czb0, ordb0 = _prep(b0)
        d2b, cxb1, cyb1, czb1, ordb1 = _prep(b1)

        def sel_body(k, carry):
            mnp0, idp0, ia0, mnp1, idp1, ia1 = carry
            mn0, idx0 = _sel_step(d2a, mnp0, idp0)
            mn1, idx1 = _sel_step(d2b, mnp1, idp1)
            colk = iota_gk == k
            ia0 = jnp.where(colk, idx0, ia0)
            ia1 = jnp.where(colk, idx1, ia1)
            return mn0, idx0, ia0, mn1, idx1, ia1

        mninit = jnp.full((G, 1), -_INF, dtype=f32)
        idinit = jnp.full((G, 1), -1, dtype=jnp.int32)
        iainit = jnp.zeros((G, K), dtype=jnp.int32)
        _, _, ia0, _, _, ia1 = jax.lax.fori_loop(
            0, K, sel_body,
            (mninit, idinit, iainit, mninit, idinit, iainit))
        _finish(b0, ia0, cxb0, cyb0, czb0, ordb0)
        _finish(b1, ia1, cxb1, cyb1, czb1, ordb1)
        return 0

    jax.lax.fori_loop(0, B // 2, knn_pair, 0)


def _sc_gather_body(xyzp_hbm, idx_hbm, cent_hbm, out_hbm,
                    idx_v, rows_v, cent_v, sem):
    wid = jax.lax.axis_index("s") * _SC_NC + jax.lax.axis_index("c")
    base = wid * _RPT
    pltpu.sync_copy(idx_hbm.at[pl.ds(wid * _IDX_SUB, _IDX_SUB)], idx_v)
    cps = []
    for j in range(_IDX_SUB):
        cps.append(pltpu.async_copy(xyzp_hbm.at[idx_v.at[j]],
                                    rows_v.at[pl.ds(j * 128, 128)], sem))
    for cp in cps:
        cp.wait()
    pltpu.sync_copy(cent_hbm.at[pl.ds(wid * _CPT, _CPT)], cent_v)

    def cbody(ci, _):
        cv = cent_v[ci]

        def rbody(i, __):
            r = ci * K + i
            rows_v[r] = rows_v[r] - cv
            return 0

        jax.lax.fori_loop(0, K, rbody, 0)
        return 0

    jax.lax.fori_loop(0, _CPT, cbody, 0)
    pltpu.sync_copy(rows_v, out_hbm.at[pl.ds(base, _RPT)])


@jax.jit
def kernel(xyz):
    f32 = jnp.float32
    x = xyz[:, :, 0]
    y = xyz[:, :, 1]
    z = xyz[:, :, 2]

    cxa, cya, cza = pl.pallas_call(
        _fps_kernel,
        out_shape=(
            jax.ShapeDtypeStruct((B, G), f32),
            jax.ShapeDtypeStruct((B, G), f32),
            jax.ShapeDtypeStruct((B, G), f32),
        ),
    )(x, y, z)

    center = jnp.stack([cxa, cya, cza], axis=-1)
    a2 = jnp.sum(center * center, axis=-1)
    ab = jnp.einsum('bmd,bnd->bmn', center, center)
    dmat = a2[:, :, None] + a2[:, None, :] - 2.0 * ab

    idxp, cxp, cyp, czp = pl.pallas_call(
        _group_kernel,
        out_shape=(
            jax.ShapeDtypeStruct((B, G, K), jnp.int32),
            jax.ShapeDtypeStruct((B, G), f32),
            jax.ShapeDtypeStruct((B, G), f32),
            jax.ShapeDtypeStruct((B, G), f32),
        ),
    )(x, y, z, cxa, cya, cza, dmat)

    centp = jnp.stack([cxp, cyp, czp], axis=-1)
    xyzp = jnp.pad(xyz.reshape(B * N, 3), ((0, 0), (0, 13)))
    cent16 = jnp.pad(centp.reshape(B * G, 3), ((0, 0), (0, 13)))
    idx2d = idxp.reshape(_SC_NW * _IDX_SUB, 128)

    nb16 = pl.kernel(
        _sc_gather_body,
        out_type=jax.ShapeDtypeStruct((_ROWS, 16), f32),
        mesh=plsc.VectorSubcoreMesh(core_axis_name="c", subcore_axis_name="s"),
        scratch_types=[
            pltpu.VMEM((_IDX_SUB, 128), jnp.int32),
            pltpu.VMEM((_RPT, 16), f32),
            pltpu.VMEM((_CPT, 16), f32),
            pltpu.SemaphoreType.DMA,
        ],
        compiler_params=pltpu.CompilerParams(use_tc_tiling_on_sc=False),
    )(xyzp, idx2d, cent16)

    neighborhood = nb16.reshape(B, G, K, 16)[:, :, :, 0:3]
    return neighborhood, centp
